# baseline (speedup 1.0000x reference)
import functools

import jax
import jax.numpy as jnp
import numpy as np
from jax.experimental import pallas as pl

B, T, H, W = 2, 2, 64, 256
FOV_UP = 3.0 * np.pi / 180.0
FOV_DOWN = -25.0 * np.pi / 180.0
MASK_THRESHOLD = 0.5
BT = B * T
N = H * W + 1
NPAD = 16896
TQ = 1056
TR = 4224
NQT = NPAD // TQ
NRT = NPAD // TR
NPROB = 2 * BT


def _chamfer_body(qn_ref, rt_ref, s_ref, c_ref):
    q = pl.program_id(1)
    qb = qn_ref[0]
    qx = qb[:, 0:1]
    qy = qb[:, 1:2]
    qz = qb[:, 2:3]
    m128 = jnp.full((TQ, 128), jnp.inf, jnp.float32)
    for t in range(NRT):
        r = rt_ref[0, :, pl.ds(t * TR, TR)]
        dx = qx - r[0:1]
        dy = qy - r[1:2]
        dz = qz - r[2:3]
        v = dx * dx + dy * dy + dz * dz
        m128 = jnp.minimum(m128, jnp.min(v.reshape(TQ, TR // 128, 128), axis=1))
    m = jnp.min(m128, axis=1, keepdims=True)
    s = jnp.sum(m)
    c = jnp.sum((m > 0.0).astype(jnp.float32))
    sv = jnp.full((1, 1, 128), s, jnp.float32)
    cv = jnp.full((1, 1, 128), c, jnp.float32)

    @pl.when(q == 0)
    def _():
        s_ref[...] = sv
        c_ref[...] = cv

    @pl.when(q != 0)
    def _():
        s_ref[...] = s_ref[...] + sv
        c_ref[...] = c_ref[...] + cv


@functools.partial(jax.jit)
def _chamfer(output_rv, output_mask_logits, target):
    mask_prob = jax.nn.sigmoid(output_mask_logits)
    masked_rv = jnp.where(mask_prob > MASK_THRESHOLD, output_rv, -1.0)
    rv = masked_rv.reshape(BT, H, W)

    h = jnp.arange(H, dtype=jnp.float32)
    w = jnp.arange(W, dtype=jnp.float32)
    yaw = -((w + 0.5) / W * 2.0 - 1.0) * jnp.pi
    pitch = (1.0 - (h + 0.5) / H) * (FOV_UP - FOV_DOWN) + FOV_DOWN
    yaw2 = jnp.broadcast_to(yaw[None, :], (H, W))
    pitch2 = jnp.broadcast_to(pitch[:, None], (H, W))
    x = rv * (jnp.cos(pitch2) * jnp.cos(yaw2))[None]
    y = rv * (jnp.cos(pitch2) * jnp.sin(yaw2))[None]
    z = rv * jnp.sin(pitch2)[None]
    valid = rv > 0.0
    ox = jnp.where(valid, x, 1000.0).reshape(BT, H * W)
    oy = jnp.where(valid, y, 1000.0).reshape(BT, H * W)
    oz = jnp.where(valid, z, 1000.0).reshape(BT, H * W)

    tvalid = (target[:, :, 0] >= 0.0).reshape(BT, H * W)
    tx = jnp.where(tvalid, target[:, :, 1].reshape(BT, H * W), 1000.0)
    ty = jnp.where(tvalid, target[:, :, 2].reshape(BT, H * W), 1000.0)
    tz = jnp.where(tvalid, target[:, :, 3].reshape(BT, H * W), 1000.0)

    def build(cx, cy, cz, axis):
        cx = jnp.pad(cx, ((0, 0), (0, NPAD - H * W)), constant_values=1000.0)
        cy = jnp.pad(cy, ((0, 0), (0, NPAD - H * W)), constant_values=1000.0)
        cz = jnp.pad(cz, ((0, 0), (0, NPAD - H * W)), constant_values=1000.0)
        zero = jnp.zeros_like(cx)
        return jnp.stack([cx, cy, cz, zero, zero, zero, zero, zero], axis=axis)

    qn = jnp.concatenate([build(ox, oy, oz, -1),
                          build(tx, ty, tz, -1)], axis=0)
    rt = jnp.concatenate([build(tx, ty, tz, 1),
                          build(ox, oy, oz, 1)], axis=0)

    s, c = pl.pallas_call(
        _chamfer_body,
        grid=(NPROB, NQT),
        in_specs=[
            pl.BlockSpec((1, TQ, 8), lambda p, q: (p, q, 0)),
            pl.BlockSpec((1, 8, NPAD), lambda p, q: (p, 0, 0)),
        ],
        out_specs=[
            pl.BlockSpec((1, 1, 128), lambda p, q: (p, 0, 0)),
            pl.BlockSpec((1, 1, 128), lambda p, q: (p, 0, 0)),
        ],
        out_shape=[
            jax.ShapeDtypeStruct((NPROB, 1, 128), jnp.float32),
            jax.ShapeDtypeStruct((NPROB, 1, 128), jnp.float32),
        ],
    )(qn, rt)
    s = s[:, 0, 0]
    c = c[:, 0, 0]

    dist_combined = s[:BT] / c[:BT] + s[BT:] / c[BT:]
    chamfer_distances_tensor = dist_combined.reshape(T, B)
    chamf_dist_t = jnp.mean(chamfer_distances_tensor, axis=1)
    return chamf_dist_t, chamfer_distances_tensor


def kernel(output_rv, output_mask_logits, target):
    return _chamfer(output_rv, output_mask_logits, target)
